# baseline (speedup 1.0000x reference)
import functools

import jax
import jax.numpy as jnp
from jax import lax
from jax.experimental import pallas as pl
from jax.experimental.pallas import tpu as pltpu
from jax.experimental.pallas import tpu_sc as plsc

_B = 16384
_E = 64
_NC = 2
_NS = 16
_NW = _NC * _NS
_BPW = _B // _NW
_CH = 128
_NCH = _BPW // _CH


def _body(user_h, course_h, ut_h, ct_h, wb_h, out_h,
          uidx, cidx, urows, crows, wbv, outv, *sems):
    cid = lax.axis_index("c")
    sid = lax.axis_index("s")
    wid = sid * _NC + cid
    base = wid * _BPW

    pltpu.sync_copy(wb_h, wbv)
    pltpu.sync_copy(user_h.at[pl.ds(base, _BPW)], uidx)
    pltpu.sync_copy(course_h.at[pl.ds(base, _BPW)], cidx)

    hu, hc = [], []
    for ch in range(_NCH):
        sl = pl.ds(ch * _CH, _CH)
        hu.append(pltpu.async_copy(ut_h.at[uidx.at[sl]], urows.at[sl],
                                   sems[ch]))
        hc.append(pltpu.async_copy(ct_h.at[cidx.at[sl]], crows.at[sl],
                                   sems[_NCH + ch]))

    wvecs = [wbv[pl.ds(k * 16, 16)] for k in range(5)]
    bias = wvecs[4][0]
    for ch in range(_NCH):
        hu[ch].wait()
        hc[ch].wait()

        def g_body(g, carry, ch=ch):
            rows = ch * _CH + g * 16 + lax.iota(jnp.int32, 16)
            acc = jnp.zeros((16,), jnp.float32) + bias
            for j in range(_E):
                jv = jnp.full((16,), j, jnp.int32)
                uu = plsc.load_gather(urows, [rows, jv])
                cc = plsc.load_gather(crows, [rows, jv])
                acc = acc + uu * cc * wvecs[j // 16][j % 16]
            off = pl.multiple_of(ch * _CH + g * 16, 16)
            outv[pl.ds(off, 16)] = acc
            return carry

        lax.fori_loop(0, _CH // 16, g_body, 0)

    pltpu.sync_copy(outv, out_h.at[pl.ds(base, _BPW)])


@jax.jit
def _run(user, course, user_table, course_table, wb):
    mesh = plsc.VectorSubcoreMesh(core_axis_name="c", subcore_axis_name="s")
    f = pl.kernel(
        _body,
        mesh=mesh,
        compiler_params=pltpu.CompilerParams(
            needs_layout_passes=False, use_tc_tiling_on_sc=False),
        out_type=jax.ShapeDtypeStruct((_B,), jnp.float32),
        scratch_types=[
            pltpu.VMEM((_BPW,), jnp.int32),
            pltpu.VMEM((_BPW,), jnp.int32),
            pltpu.VMEM((_BPW, _E), jnp.float32),
            pltpu.VMEM((_BPW, _E), jnp.float32),
            pltpu.VMEM((80,), jnp.float32),
            pltpu.VMEM((_BPW,), jnp.float32),
        ] + [pltpu.SemaphoreType.DMA] * (2 * _NCH),
    )
    return f(user, course, user_table, course_table, wb)


def kernel(user, course, user_table, course_table, fc_w, fc_b):
    wb = jnp.zeros((80,), jnp.float32)
    wb = wb.at[:_E].set(fc_w.reshape(-1)).at[_E].set(fc_b[0])
    out = _run(user, course, user_table, course_table, wb)
    return out.reshape(_B, 1)


# speedup vs baseline: 1.6032x; 1.6032x over previous
import jax
import jax.numpy as jnp
from jax import lax
from jax.experimental import pallas as pl
from jax.experimental.pallas import tpu as pltpu
from jax.experimental.pallas import tpu_sc as plsc

_B = 16384
_E = 64
_NC = 2
_NS = 16
_NW = _NC * _NS
_BPW = _B // _NW
_CH = 128
_NCH = _BPW // _CH


def _body(user_h, course_h, ut_h, ct_h, wb_h, out_h,
          uidx, cidx, u0, u1, c0, c1, wbv, outv, *sems):
    cid = lax.axis_index("c")
    sid = lax.axis_index("s")
    wid = sid * _NC + cid
    base = wid * _BPW

    ubufs = (u0, u1)
    cbufs = (c0, c1)

    pltpu.sync_copy(wb_h, wbv)
    pltpu.sync_copy(user_h.at[pl.ds(base, _BPW)], uidx)
    pltpu.sync_copy(course_h.at[pl.ds(base, _BPW)], cidx)

    def issue_chunk(ch):
        ub = ubufs[ch % 2]
        cb = cbufs[ch % 2]

        def g_body(g, carry):
            off = ch * _CH + g * 16
            iu = uidx[pl.ds(off, 16)]
            ic = cidx[pl.ds(off, 16)]
            dst = g * 16
            for lane in range(16):
                pltpu.async_copy(ut_h.at[pl.ds(iu[lane], 1), :],
                                 ub.at[pl.ds(dst + lane, 1), :],
                                 sems[ch])
                pltpu.async_copy(ct_h.at[pl.ds(ic[lane], 1), :],
                                 cb.at[pl.ds(dst + lane, 1), :],
                                 sems[_NCH + ch])
            return carry
        lax.fori_loop(0, _CH // 16, g_body, 0)

    def drain_chunk(ch):
        ub = ubufs[ch % 2]
        cb = cbufs[ch % 2]

        def d_body(g, carry):
            dst = g * 16
            for lane in range(16):
                pltpu.make_async_copy(
                    ut_h.at[pl.ds(0, 1), :],
                    ub.at[pl.ds(dst + lane, 1), :],
                    sems[ch]).wait()
                pltpu.make_async_copy(
                    ct_h.at[pl.ds(0, 1), :],
                    cb.at[pl.ds(dst + lane, 1), :],
                    sems[_NCH + ch]).wait()
            return carry
        lax.fori_loop(0, _CH // 16, d_body, 0)

    wvecs = [wbv[pl.ds(k * 16, 16)] for k in range(5)]
    bias = wvecs[4][0]

    def compute_chunk(ch):
        ub = ubufs[ch % 2]
        cb = cbufs[ch % 2]

        def g_body(g, carry):
            rows = g * 16 + lax.iota(jnp.int32, 16)
            acc = jnp.zeros((16,), jnp.float32) + bias
            for j in range(_E):
                jv = jnp.full((16,), j, jnp.int32)
                uu = plsc.load_gather(ub, [rows, jv])
                cc = plsc.load_gather(cb, [rows, jv])
                acc = acc + uu * cc * wvecs[j // 16][j % 16]
            off = pl.multiple_of(ch * _CH + g * 16, 16)
            outv[pl.ds(off, 16)] = acc
            return carry
        lax.fori_loop(0, _CH // 16, g_body, 0)

    issue_chunk(0)
    issue_chunk(1)
    for ch in range(_NCH):
        drain_chunk(ch)
        compute_chunk(ch)
        if ch + 2 < _NCH:
            issue_chunk(ch + 2)

    pltpu.sync_copy(outv, out_h.at[pl.ds(base, _BPW)])


@jax.jit
def _run(user, course, user_table, course_table, wb):
    mesh = plsc.VectorSubcoreMesh(core_axis_name="c", subcore_axis_name="s")
    f = pl.kernel(
        _body,
        mesh=mesh,
        compiler_params=pltpu.CompilerParams(needs_layout_passes=False),
        out_type=jax.ShapeDtypeStruct((_B,), jnp.float32),
        scratch_types=[
            pltpu.VMEM((_BPW,), jnp.int32),
            pltpu.VMEM((_BPW,), jnp.int32),
            pltpu.VMEM((_CH, _E), jnp.float32),
            pltpu.VMEM((_CH, _E), jnp.float32),
            pltpu.VMEM((_CH, _E), jnp.float32),
            pltpu.VMEM((_CH, _E), jnp.float32),
            pltpu.VMEM((80,), jnp.float32),
            pltpu.VMEM((_BPW,), jnp.float32),
        ] + [pltpu.SemaphoreType.DMA] * (2 * _NCH),
    )
    return f(user, course, user_table, course_table, wb)


def kernel(user, course, user_table, course_table, fc_w, fc_b):
    wb = jnp.zeros((80,), jnp.float32)
    wb = wb.at[:_E].set(fc_w.reshape(-1)).at[_E].set(fc_b[0])
    out = _run(user, course, user_table, course_table, wb)
    return out.reshape(_B, 1)
